# baseline (speedup 1.0000x reference)
import functools

import jax
import jax.numpy as jnp
from jax import lax
from jax.experimental import pallas as pl
from jax.experimental.pallas import tpu as pltpu
from jax.experimental.pallas import tpu_sc as plsc

_NRES = 5
_NUM = 4


def _rup(a, m):
    return (a + m - 1) // m * m


def _padw(c):
    return _rup(c, 128)


def _pad_cols(x, p):
    if x.shape[1] == p:
        return x
    return jnp.pad(x, ((0, 0), (0, p - x.shape[1])))


def _gather_rows(table, idx):
    V, D = table.shape
    B = idx.shape[0]
    info = plsc.get_sparse_core_info()
    NC, NSUB = info.num_cores, info.num_subcores
    NW = NC * NSUB
    if B >= 128 * NW:
        Bp = _rup(B, 128 * NW)
        CHK = 128
    else:
        Bp = _rup(B, 8 * NW)
        CHK = Bp // NW
    idx_p = jnp.pad(idx, (0, Bp - B))
    bpw = Bp // NW
    nch = bpw // CHK

    @functools.partial(
        pl.kernel,
        mesh=plsc.VectorSubcoreMesh(core_axis_name="c", subcore_axis_name="s"),
        out_type=jax.ShapeDtypeStruct((Bp, D), jnp.float32),
        scratch_types=[
            pltpu.VMEM((CHK,), jnp.int32),
            pltpu.VMEM((CHK, D), jnp.float32),
            pltpu.SemaphoreType.DMA,
        ],
    )
    def k(table_hbm, idx_hbm, out_hbm, idx_v, rows_v, sem):
        wid = lax.axis_index("s") * NC + lax.axis_index("c")
        base = wid * bpw

        def body(i, c):
            off = base + i * CHK
            pltpu.sync_copy(idx_hbm.at[pl.ds(off, CHK)], idx_v)
            pltpu.async_copy(table_hbm.at[idx_v], rows_v, sem).wait()
            pltpu.sync_copy(rows_v, out_hbm.at[pl.ds(off, CHK)])
            return c

        lax.fori_loop(0, nch, body, 0)

    return k(table, idx_p)[:B]


def _linear(x, wt, b):
    m, d = x.shape
    oc = wt.shape[1]
    TM = min(512, _rup(m, 8))
    mp = _rup(m, TM)
    xp = jnp.pad(x, ((0, mp - m), (0, 0)))
    b2 = b.reshape(1, oc)

    def body(x_ref, w_ref, b_ref, o_ref):
        o_ref[:] = (
            jnp.dot(x_ref[:], w_ref[:], preferred_element_type=jnp.float32)
            + b_ref[:]
        )

    out = pl.pallas_call(
        body,
        grid=(mp // TM,),
        in_specs=[
            pl.BlockSpec((TM, d), lambda i: (i, 0)),
            pl.BlockSpec((d, oc), lambda i: (0, 0)),
            pl.BlockSpec((1, oc), lambda i: (0, 0)),
        ],
        out_specs=pl.BlockSpec((TM, oc), lambda i: (i, 0)),
        out_shape=jax.ShapeDtypeStruct((mp, oc), jnp.float32),
    )(xp, wt, b2)
    return out[:m]


def _bn_lrelu(y, g, be):
    n, c = y.shape
    TM = min(2048, _rup(n, 8))
    npad = _rup(n, TM)
    yp = jnp.pad(y, ((0, npad - n), (0, 0)))
    nt = npad // TM

    def body(y_ref, g_ref, b_ref, o_ref, stat_ref):
        p = pl.program_id(0)
        i = pl.program_id(1)
        yv = y_ref[:]
        rows = lax.broadcasted_iota(jnp.int32, (TM, c), 0) + i * TM
        mask = rows < n

        @pl.when(jnp.logical_and(p == 0, i == 0))
        def _init():
            stat_ref[:] = jnp.zeros((8, c), jnp.float32)

        @pl.when(p == 0)
        def _acc():
            ym = jnp.where(mask, yv, 0.0)
            stat_ref[0:1, :] += jnp.sum(ym, axis=0, keepdims=True)
            stat_ref[1:2, :] += jnp.sum(ym * yv, axis=0, keepdims=True)
            o_ref[:] = yv

        @pl.when(p == 1)
        def _norm():
            cnt = jnp.float32(n)
            mu = stat_ref[0:1, :] / cnt
            var = stat_ref[1:2, :] / cnt - mu * mu
            xn = (yv - mu) * lax.rsqrt(var + 1e-5) * g_ref[:] + b_ref[:]
            o_ref[:] = jnp.where(xn >= 0, xn, 0.2 * xn)

    out = pl.pallas_call(
        body,
        grid=(2, nt),
        in_specs=[
            pl.BlockSpec((TM, c), lambda p, i: (i, 0)),
            pl.BlockSpec((1, c), lambda p, i: (0, 0)),
            pl.BlockSpec((1, c), lambda p, i: (0, 0)),
        ],
        out_specs=pl.BlockSpec((TM, c), lambda p, i: (i, 0)),
        out_shape=jax.ShapeDtypeStruct((npad, c), jnp.float32),
        scratch_shapes=[pltpu.VMEM((8, c), jnp.float32)],
    )(yp, g.reshape(1, c), be.reshape(1, c))
    return out[:n]


def _conv_wt(W, ic, icp):
    oc = W.shape[0]
    w3 = W.reshape(oc, 7, ic)
    w3 = jnp.pad(w3, ((0, 0), (0, 0), (0, icp - ic)))
    return w3.reshape(oc, 7 * icp).T


def _onering(x_real, neigh, W, b):
    n, ic = x_real.shape
    icp = _padw(ic)
    xp = _pad_cols(x_real, icp)
    g = _gather_rows(xp, neigh)
    mat = g.reshape(n, 7 * icp)
    return _linear(mat, _conv_wt(W, ic, icp), b)


def _dconv(x_real, neigh, p):
    y = _onering(x_real, neigh, p["W1"], p["b1"])
    y = _bn_lrelu(y, p["g1"], p["be1"])
    z = _onering(y, neigh, p["W2"], p["b2"])
    return _bn_lrelu(z, p["g2"], p["be2"])


def _pool(x_real, neigh):
    num = (x_real.shape[0] + 6) // 4
    f = x_real.shape[1]
    fp = _padw(f)
    xp = _pad_cols(x_real, fp)
    g = _gather_rows(xp, neigh[: num * 7])
    mat = g.reshape(num, 7 * fp)
    P = jnp.zeros((7 * f, f), jnp.float32)
    ii = jnp.arange(7 * f)
    P = P.at[ii, ii // 7].set(1.0 / 7.0)
    P = jnp.pad(P.reshape(7, f, f), ((0, 0), (0, fp - f), (0, 0))).reshape(
        7 * fp, f
    )
    return _linear(mat, P, jnp.zeros((f,), jnp.float32))


def _upconv(x_real, Wu, bu, down):
    raw, ic = x_real.shape
    oc = Wu.shape[0] // 7
    ocp = _padw(oc)
    icp = _padw(ic)
    xp = _pad_cols(x_real, icp)
    wu3 = jnp.pad(
        Wu.reshape(7, oc, ic), ((0, 0), (0, ocp - oc), (0, icp - ic))
    )
    wut = wu3.reshape(7 * ocp, icp).T
    bu2 = jnp.pad(bu.reshape(7, oc), ((0, 0), (0, ocp - oc))).reshape(-1)
    y = _linear(xp, wut, bu2)
    y1 = y[:, :oc]
    yflat = y.reshape(raw * 7, ocp)
    g = _gather_rows(yflat, down)
    M = down.shape[0] // 2
    h = oc // 2
    H = jnp.zeros((oc, h), jnp.float32)
    jj = jnp.arange(oc)
    H = H.at[jj, jj // 2].set(0.5)
    H = jnp.pad(H, ((0, ocp - oc), (0, 0)))
    pm = _linear(g, H, jnp.zeros((h,), jnp.float32))
    y2 = pm.reshape(M, oc)
    return jnp.concatenate([y1, y2], axis=0)


def _up_block(x1, x2, p, neigh, down):
    x1 = _upconv(x1, p["Wu"], p["bu"], down)
    x = jnp.concatenate([x1, x2], axis=1)
    return _dconv(x, neigh, p)


def _down_all(x0, params, idx):
    xs = [x0]
    for i in range(_NRES):
        x = xs[i]
        if i > 0:
            x = _pool(x, idx["neigh"][i - 1])
        x = _dconv(x, idx["neigh"][i], params["down"][i])
        xs.append(x)
    return xs


def _proj(u, W, b):
    ic = u.shape[1]
    icp = _padw(ic)
    return _linear(_pad_cols(u, icp), jnp.pad(W.T, ((0, icp - ic), (0, 0))), b)


def kernel(x, x2, params, idx):
    xis = [_down_all(x[i], params, idx) for i in range(_NUM)]
    xs_parc = []
    for i in range(_NUM):
        xs = xis[i]
        y = xs[-1]
        for j in range(_NRES - 1):
            k = _NRES - 2 - j
            y = _up_block(
                y, xs[_NRES - 1 - j], params["up_parc"][j], idx["neigh"][k],
                idx["down"][k],
            )
        xs_parc.append(_proj(y, params["Wp"], params["bp"]))
    x2s = _down_all(x2, params, idx)
    z = x2s[-1]
    for j in range(_NRES - 1):
        k = _NRES - 2 - j
        z = _up_block(
            z, x2s[_NRES - 1 - j], params["up_parc"][j], idx["neigh"][k],
            idx["down"][k],
        )
    x2_parc = _proj(z, params["Wp"], params["bp"])
    feats = []
    for i in range(_NRES + 1):
        tmp = x2s[i]
        for j in range(_NUM):
            tmp = jnp.concatenate([tmp, xis[j][i]], axis=1)
        feats.append(tmp)
    u = feats[-1]
    for i in range(_NRES - 1):
        k = _NRES - 2 - i
        u = _up_block(
            u, feats[_NRES - 1 - i], params["up_reg"][i], idx["neigh"][k],
            idx["down"][k],
        )
    x_reg = _proj(u, params["Wr"], params["br"])
    return (*xs_parc, x2_parc, x_reg)


# speedup vs baseline: 1.6613x; 1.6613x over previous
import functools

import jax
import jax.numpy as jnp
from jax import lax
from jax.experimental import pallas as pl
from jax.experimental.pallas import tpu as pltpu
from jax.experimental.pallas import tpu_sc as plsc

_NRES = 5
_NUM = 4


def _rup(a, m):
    return (a + m - 1) // m * m


def _padw(c):
    return _rup(c, 16)


def _pad_cols(x, p):
    if x.shape[1] == p:
        return x
    return jnp.pad(x, ((0, 0), (0, p - x.shape[1])))


def _gather_rows(table, idx):
    V, D = table.shape
    B = idx.shape[0]
    info = plsc.get_sparse_core_info()
    NC, NSUB = info.num_cores, info.num_subcores
    NW = NC * NSUB
    if B >= 128 * NW:
        Bp = _rup(B, 128 * NW)
        CHK = 128
    else:
        Bp = _rup(B, 8 * NW)
        CHK = Bp // NW
    idx_p = jnp.pad(idx, (0, Bp - B))
    bpw = Bp // NW
    nch = bpw // CHK

    @functools.partial(
        pl.kernel,
        mesh=plsc.VectorSubcoreMesh(core_axis_name="c", subcore_axis_name="s"),
        out_type=jax.ShapeDtypeStruct((Bp, D), jnp.float32),
        scratch_types=[
            pltpu.VMEM((CHK,), jnp.int32),
            pltpu.VMEM((CHK, D), jnp.float32),
            pltpu.SemaphoreType.DMA,
        ],
        compiler_params=pltpu.CompilerParams(use_tc_tiling_on_sc=False),
    )
    def k(table_hbm, idx_hbm, out_hbm, idx_v, rows_v, sem):
        wid = lax.axis_index("s") * NC + lax.axis_index("c")
        base = wid * bpw

        def body(i, c):
            off = base + i * CHK
            pltpu.sync_copy(idx_hbm.at[pl.ds(off, CHK)], idx_v)
            pltpu.async_copy(table_hbm.at[idx_v], rows_v, sem).wait()
            pltpu.sync_copy(rows_v, out_hbm.at[pl.ds(off, CHK)])
            return c

        lax.fori_loop(0, nch, body, 0)

    return k(table, idx_p)[:B]


def _linear(x, wt, b):
    m, d = x.shape
    oc = wt.shape[1]
    TM = min(512, _rup(m, 8))
    mp = _rup(m, TM)
    xp = jnp.pad(x, ((0, mp - m), (0, 0)))
    b2 = b.reshape(1, oc)

    def body(x_ref, w_ref, b_ref, o_ref):
        o_ref[:] = (
            jnp.dot(x_ref[:], w_ref[:], preferred_element_type=jnp.float32)
            + b_ref[:]
        )

    out = pl.pallas_call(
        body,
        grid=(mp // TM,),
        in_specs=[
            pl.BlockSpec((TM, d), lambda i: (i, 0)),
            pl.BlockSpec((d, oc), lambda i: (0, 0)),
            pl.BlockSpec((1, oc), lambda i: (0, 0)),
        ],
        out_specs=pl.BlockSpec((TM, oc), lambda i: (i, 0)),
        out_shape=jax.ShapeDtypeStruct((mp, oc), jnp.float32),
    )(xp, wt, b2)
    return out[:m]


def _bn_lrelu(y, g, be):
    n, c = y.shape
    TM = min(2048, _rup(n, 8))
    npad = _rup(n, TM)
    yp = jnp.pad(y, ((0, npad - n), (0, 0)))
    nt = npad // TM

    def body(y_ref, g_ref, b_ref, o_ref, stat_ref):
        p = pl.program_id(0)
        i = pl.program_id(1)
        yv = y_ref[:]
        rows = lax.broadcasted_iota(jnp.int32, (TM, c), 0) + i * TM
        mask = rows < n

        @pl.when(jnp.logical_and(p == 0, i == 0))
        def _init():
            stat_ref[:] = jnp.zeros((8, c), jnp.float32)

        @pl.when(p == 0)
        def _acc():
            ym = jnp.where(mask, yv, 0.0)
            stat_ref[0:1, :] += jnp.sum(ym, axis=0, keepdims=True)
            stat_ref[1:2, :] += jnp.sum(ym * yv, axis=0, keepdims=True)
            o_ref[:] = yv

        @pl.when(p == 1)
        def _norm():
            cnt = jnp.float32(n)
            mu = stat_ref[0:1, :] / cnt
            var = stat_ref[1:2, :] / cnt - mu * mu
            xn = (yv - mu) * lax.rsqrt(var + 1e-5) * g_ref[:] + b_ref[:]
            o_ref[:] = jnp.where(xn >= 0, xn, 0.2 * xn)

    out = pl.pallas_call(
        body,
        grid=(2, nt),
        in_specs=[
            pl.BlockSpec((TM, c), lambda p, i: (i, 0)),
            pl.BlockSpec((1, c), lambda p, i: (0, 0)),
            pl.BlockSpec((1, c), lambda p, i: (0, 0)),
        ],
        out_specs=pl.BlockSpec((TM, c), lambda p, i: (i, 0)),
        out_shape=jax.ShapeDtypeStruct((npad, c), jnp.float32),
        scratch_shapes=[pltpu.VMEM((8, c), jnp.float32)],
    )(yp, g.reshape(1, c), be.reshape(1, c))
    return out[:n]


def _conv_wt(W, ic, icp):
    oc = W.shape[0]
    w3 = W.reshape(oc, 7, ic)
    w3 = jnp.pad(w3, ((0, 0), (0, 0), (0, icp - ic)))
    return w3.reshape(oc, 7 * icp).T


def _onering(x_real, neigh, W, b):
    n, ic = x_real.shape
    icp = _padw(ic)
    xp = _pad_cols(x_real, icp)
    g = _gather_rows(xp, neigh)
    mat = g.reshape(n, 7 * icp)
    return _linear(mat, _conv_wt(W, ic, icp), b)


def _dconv(x_real, neigh, p):
    y = _onering(x_real, neigh, p["W1"], p["b1"])
    y = _bn_lrelu(y, p["g1"], p["be1"])
    z = _onering(y, neigh, p["W2"], p["b2"])
    return _bn_lrelu(z, p["g2"], p["be2"])


def _pool(x_real, neigh):
    num = (x_real.shape[0] + 6) // 4
    f = x_real.shape[1]
    fp = _padw(f)
    xp = _pad_cols(x_real, fp)
    g = _gather_rows(xp, neigh[: num * 7])
    mat = g.reshape(num, 7 * fp)
    P = jnp.zeros((7 * f, f), jnp.float32)
    ii = jnp.arange(7 * f)
    P = P.at[ii, ii // 7].set(1.0 / 7.0)
    P = jnp.pad(P.reshape(7, f, f), ((0, 0), (0, fp - f), (0, 0))).reshape(
        7 * fp, f
    )
    return _linear(mat, P, jnp.zeros((f,), jnp.float32))


def _upconv(x_real, Wu, bu, down):
    raw, ic = x_real.shape
    oc = Wu.shape[0] // 7
    ocp = _padw(oc)
    icp = _padw(ic)
    xp = _pad_cols(x_real, icp)
    wu3 = jnp.pad(
        Wu.reshape(7, oc, ic), ((0, 0), (0, ocp - oc), (0, icp - ic))
    )
    wut = wu3.reshape(7 * ocp, icp).T
    bu2 = jnp.pad(bu.reshape(7, oc), ((0, 0), (0, ocp - oc))).reshape(-1)
    y = _linear(xp, wut, bu2)
    y1 = y[:, :oc]
    yflat = y.reshape(raw * 7, ocp)
    g = _gather_rows(yflat, down)
    M = down.shape[0] // 2
    h = oc // 2
    H = jnp.zeros((oc, h), jnp.float32)
    jj = jnp.arange(oc)
    H = H.at[jj, jj // 2].set(0.5)
    H = jnp.pad(H, ((0, ocp - oc), (0, 0)))
    pm = _linear(g, H, jnp.zeros((h,), jnp.float32))
    y2 = pm.reshape(M, oc)
    return jnp.concatenate([y1, y2], axis=0)


def _up_block(x1, x2, p, neigh, down):
    x1 = _upconv(x1, p["Wu"], p["bu"], down)
    x = jnp.concatenate([x1, x2], axis=1)
    return _dconv(x, neigh, p)


def _down_all(x0, params, idx):
    xs = [x0]
    for i in range(_NRES):
        x = xs[i]
        if i > 0:
            x = _pool(x, idx["neigh"][i - 1])
        x = _dconv(x, idx["neigh"][i], params["down"][i])
        xs.append(x)
    return xs


def _proj(u, W, b):
    ic = u.shape[1]
    icp = _padw(ic)
    return _linear(_pad_cols(u, icp), jnp.pad(W.T, ((0, icp - ic), (0, 0))), b)


def kernel(x, x2, params, idx):
    xis = [_down_all(x[i], params, idx) for i in range(_NUM)]
    xs_parc = []
    for i in range(_NUM):
        xs = xis[i]
        y = xs[-1]
        for j in range(_NRES - 1):
            k = _NRES - 2 - j
            y = _up_block(
                y, xs[_NRES - 1 - j], params["up_parc"][j], idx["neigh"][k],
                idx["down"][k],
            )
        xs_parc.append(_proj(y, params["Wp"], params["bp"]))
    x2s = _down_all(x2, params, idx)
    z = x2s[-1]
    for j in range(_NRES - 1):
        k = _NRES - 2 - j
        z = _up_block(
            z, x2s[_NRES - 1 - j], params["up_parc"][j], idx["neigh"][k],
            idx["down"][k],
        )
    x2_parc = _proj(z, params["Wp"], params["bp"])
    feats = []
    for i in range(_NRES + 1):
        tmp = x2s[i]
        for j in range(_NUM):
            tmp = jnp.concatenate([tmp, xis[j][i]], axis=1)
        feats.append(tmp)
    u = feats[-1]
    for i in range(_NRES - 1):
        k = _NRES - 2 - i
        u = _up_block(
            u, feats[_NRES - 1 - i], params["up_reg"][i], idx["neigh"][k],
            idx["down"][k],
        )
    x_reg = _proj(u, params["Wr"], params["br"])
    return (*xs_parc, x2_parc, x_reg)
